# baseline (speedup 1.0000x reference)
import jax
import jax.numpy as jnp
from jax import lax
from jax.experimental import pallas as pl
from jax.experimental.pallas import tpu as pltpu

S = 128
K = 8
RADIUS = 1.5
R2 = (2.0 * RADIUS / S) ** 2
INV_R2 = 1.0 / R2
ROWS = 2
PIX = ROWS * S
NBANDS = S // ROWS
SEG = 4096
W = 272
SW = SEG + W
WIDE = 512
EPS = 1e-6


def _raster_body(starts_ref, coords_ref, coordst_ref, feats_ref, out_ref):
    t = pl.program_id(1)
    c_dim = feats_ref.shape[2]
    nb = starts_ref.shape[3]
    pidx = lax.broadcasted_iota(jnp.int32, (1, PIX), 1)
    rowi = t * ROWS + pidx // S
    coli = pidx - (pidx // S) * S
    cx = 1.0 - (2.0 * coli.astype(jnp.float32) + 1.0) / S
    cy = 1.0 - (2.0 * rowi.astype(jnp.float32) + 1.0) / S

    log_t = jnp.zeros((1, PIX), jnp.float32)
    cnt = jnp.zeros((1, PIX), jnp.float32)
    acc = jnp.zeros((c_dim, PIX), jnp.float32)
    for bk in range(nb):
        s = pl.multiple_of(starts_ref[0, 0, 0, bk], 8)
        s128 = pl.multiple_of((s // 128) * 128, 128)
        d = s - s128
        blk = coords_ref[0, pl.ds(s, W), :]
        fblk = feats_ref[0, pl.ds(s, W), :]
        xcol = blk[:, 0:1]
        ycol = blk[:, 1:2]
        zcol = blk[:, 2:3]
        icol = blk[:, 3:4]
        ziwide = coordst_ref[0, 2:4, pl.ds(s128, WIDE)]
        zirot = pltpu.roll(ziwide, -d, axis=1)
        zrow = zirot[0:1, :W]
        irow = zirot[1:2, :W]
        d2 = (xcol - cx) ** 2 + (ycol - cy) ** 2
        m = d2 <= R2
        asq = jnp.sqrt(jnp.maximum(d2 * INV_R2, 0.001))
        l1a = jnp.where(m, 0.5 * jnp.log(jnp.maximum(d2 * INV_R2, 0.001)),
                        0.0)
        mf = m.astype(jnp.float32)
        x2 = jnp.concatenate([l1a, mf], axis=1)
        cmp = jnp.where((zcol < zrow) | ((zcol == zrow) & (icol < irow)),
                        1.0, 0.0)
        e = lax.dot_general(cmp, x2, (((0,), (0,)), ((), ())),
                            preferred_element_type=jnp.float32)
        w = jnp.where(m & (e[:, PIX:] + cnt < (K - 0.5)),
                      (1.0 - asq) * jnp.exp(e[:, :PIX] + log_t), 0.0)
        acc = acc + lax.dot_general(
            fblk, w, (((0,), (0,)), ((), ())),
            preferred_element_type=jnp.float32)
        log_t = log_t + jnp.sum(l1a, axis=0, keepdims=True)
        cnt = cnt + jnp.sum(mf, axis=0, keepdims=True)
    out_ref[0] = acc


def _pad_segments(arr, nb, nt, fill):
    b = arr.shape[0]
    trail = arr.shape[2:]
    a4 = arr.reshape(b, nb, SEG, *trail)
    padv = jnp.full((b, nb, W, *trail), fill, arr.dtype)
    a4 = jnp.concatenate([a4, padv], axis=2).reshape(b, nb * SW, *trail)
    tail = jnp.full((b, nt - nb * SW, *trail), fill, arr.dtype)
    return jnp.concatenate([a4, tail], axis=1)


def kernel(pts, features):
    b, n, _ = pts.shape
    c_dim = features.shape[-1]
    neg = pts * jnp.array([-1.0, -1.0, 1.0], dtype=pts.dtype)
    x = neg[..., 0]
    y = neg[..., 1]
    z = neg[..., 2]
    nb = n // SEG
    nt = nb * SW + WIDE
    order_z = jnp.argsort(z, axis=-1, stable=True)
    y_zs = jnp.take_along_axis(y, order_z, axis=1)
    bucket = (jnp.arange(n, dtype=jnp.int32) // SEG).astype(jnp.float32)
    order2 = jnp.argsort(bucket[None, :] * 4.0 + y_zs, axis=-1, stable=True)
    perm = jnp.take_along_axis(order_z, order2, axis=1)
    xs = jnp.take_along_axis(x, perm, axis=1)
    ys = jnp.take_along_axis(y, perm, axis=1)
    zs = jnp.take_along_axis(z, perm, axis=1)
    permf = perm.astype(jnp.float32)
    xp = _pad_segments(xs, nb, nt, 1e9)
    yp = _pad_segments(ys, nb, nt, 1e9)
    zp = _pad_segments(zs, nb, nt, 1.0)
    ip = _pad_segments(permf, nb, nt, 0.0)
    coordst = jnp.stack([xp, yp, zp, ip], axis=1)
    coords = coordst.transpose(0, 2, 1)
    flat_idx = (perm + (jnp.arange(b, dtype=jnp.int32) * n)[:, None]
                ).reshape(-1)
    feats = features.reshape(b * n, c_dim)[flat_idx].reshape(b, n, c_dim)
    featsp = _pad_segments(feats, nb, nt, 0.0)
    r_eps = 2.0 * RADIUS / S + EPS
    tband = jnp.arange(NBANDS, dtype=jnp.float32)
    ylo = 1.0 - (2.0 * (tband * ROWS + ROWS - 1) + 1.0) / S - r_eps
    ys_seg = ys.reshape(b, nb, SEG)
    s_rel = (ys_seg[:, None, :, :] < ylo[None, :, None, None]).sum(
        axis=-1, dtype=jnp.int32)
    starts = ((s_rel // 8) * 8
              + (jnp.arange(nb, dtype=jnp.int32) * SW)[None, None, :])
    out = pl.pallas_call(
        _raster_body,
        grid=(b, NBANDS),
        in_specs=[
            pl.BlockSpec((1, 1, 1, nb), lambda bi, t: (bi, t, 0, 0),
                         memory_space=pltpu.SMEM),
            pl.BlockSpec((1, nt, 4), lambda bi, t: (bi, 0, 0)),
            pl.BlockSpec((1, 4, nt), lambda bi, t: (bi, 0, 0)),
            pl.BlockSpec((1, nt, c_dim), lambda bi, t: (bi, 0, 0)),
        ],
        out_specs=pl.BlockSpec((1, c_dim, PIX), lambda bi, t: (bi, 0, t)),
        out_shape=jax.ShapeDtypeStruct((b, c_dim, S * S), jnp.float32),
    )(starts.reshape(b, NBANDS, 1, nb), coords, coordst, featsp)
    return out.reshape(b, c_dim, S, S)


# speedup vs baseline: 1.1409x; 1.1409x over previous
import jax
import jax.numpy as jnp
from jax import lax
from jax.experimental import pallas as pl
from jax.experimental.pallas import tpu as pltpu

S = 128
K = 8
RADIUS = 1.5
R2 = (2.0 * RADIUS / S) ** 2
INV_R2 = 1.0 / R2
ROWS = 4
PIX = ROWS * S
NBANDS = S // ROWS
SEG = 4096
W = 344
SW = SEG + W
WIDE = 512
EPS = 1e-6


def _raster_body(starts_ref, coords_ref, coordst_ref, feats_ref, out_ref):
    t = pl.program_id(1)
    c_dim = feats_ref.shape[2]
    nb = starts_ref.shape[3]
    pidx = lax.broadcasted_iota(jnp.int32, (1, PIX), 1)
    rowi = t * ROWS + pidx // S
    coli = pidx - (pidx // S) * S
    cx = 1.0 - (2.0 * coli.astype(jnp.float32) + 1.0) / S
    cy = 1.0 - (2.0 * rowi.astype(jnp.float32) + 1.0) / S

    log_t = jnp.zeros((1, PIX), jnp.float32)
    cnt = jnp.zeros((1, PIX), jnp.float32)
    acc = jnp.zeros((c_dim, PIX), jnp.float32)
    for bk in range(nb):
        s = pl.multiple_of(starts_ref[0, 0, 0, bk], 8)
        s128 = pl.multiple_of((s // 128) * 128, 128)
        d = s - s128
        blk = coords_ref[0, pl.ds(s, W), :]
        fblk = feats_ref[0, pl.ds(s, W), :]
        xcol = blk[:, 0:1]
        ycol = blk[:, 1:2]
        zcol = blk[:, 2:3]
        icol = blk[:, 3:4]
        ziwide = coordst_ref[0, 2:4, pl.ds(s128, WIDE)]
        zirot = pltpu.roll(ziwide, -d, axis=1)
        zrow = zirot[0:1, :W]
        irow = zirot[1:2, :W]
        d2 = (xcol - cx) ** 2 + (ycol - cy) ** 2
        m = d2 <= R2
        asq = jnp.sqrt(jnp.maximum(d2 * INV_R2, 0.001))
        l1a = jnp.where(m, 0.5 * jnp.log(jnp.maximum(d2 * INV_R2, 0.001)),
                        0.0)
        mf = m.astype(jnp.float32)
        x2 = jnp.concatenate([l1a, mf], axis=1)
        cmp = jnp.where((zcol < zrow) | ((zcol == zrow) & (icol < irow)),
                        1.0, 0.0)
        e = lax.dot_general(cmp, x2, (((0,), (0,)), ((), ())),
                            preferred_element_type=jnp.float32)
        w = jnp.where(m & (e[:, PIX:] + cnt < (K - 0.5)),
                      (1.0 - asq) * jnp.exp(e[:, :PIX] + log_t), 0.0)
        acc = acc + lax.dot_general(
            fblk, w, (((0,), (0,)), ((), ())),
            preferred_element_type=jnp.float32)
        log_t = log_t + jnp.sum(l1a, axis=0, keepdims=True)
        cnt = cnt + jnp.sum(mf, axis=0, keepdims=True)
    out_ref[0] = acc


def _pad_segments(arr, nb, nt, fill):
    b = arr.shape[0]
    trail = arr.shape[2:]
    a4 = arr.reshape(b, nb, SEG, *trail)
    padv = jnp.full((b, nb, W, *trail), fill, arr.dtype)
    a4 = jnp.concatenate([a4, padv], axis=2).reshape(b, nb * SW, *trail)
    tail = jnp.full((b, nt - nb * SW, *trail), fill, arr.dtype)
    return jnp.concatenate([a4, tail], axis=1)


def kernel(pts, features):
    b, n, _ = pts.shape
    c_dim = features.shape[-1]
    neg = pts * jnp.array([-1.0, -1.0, 1.0], dtype=pts.dtype)
    x = neg[..., 0]
    y = neg[..., 1]
    z = neg[..., 2]
    nb = n // SEG
    nt = nb * SW + WIDE
    order_z = jnp.argsort(z, axis=-1, stable=True)
    y_zs = jnp.take_along_axis(y, order_z, axis=1)
    bucket = (jnp.arange(n, dtype=jnp.int32) // SEG).astype(jnp.float32)
    order2 = jnp.argsort(bucket[None, :] * 4.0 + y_zs, axis=-1, stable=True)
    perm = jnp.take_along_axis(order_z, order2, axis=1)
    xs = jnp.take_along_axis(x, perm, axis=1)
    ys = jnp.take_along_axis(y, perm, axis=1)
    zs = jnp.take_along_axis(z, perm, axis=1)
    permf = perm.astype(jnp.float32)
    xp = _pad_segments(xs, nb, nt, 1e9)
    yp = _pad_segments(ys, nb, nt, 1e9)
    zp = _pad_segments(zs, nb, nt, 1.0)
    ip = _pad_segments(permf, nb, nt, 0.0)
    coordst = jnp.stack([xp, yp, zp, ip], axis=1)
    coords = coordst.transpose(0, 2, 1)
    flat_idx = (perm + (jnp.arange(b, dtype=jnp.int32) * n)[:, None]
                ).reshape(-1)
    feats = features.reshape(b * n, c_dim)[flat_idx].reshape(b, n, c_dim)
    featsp = _pad_segments(feats, nb, nt, 0.0)
    r_eps = 2.0 * RADIUS / S + EPS
    tband = jnp.arange(NBANDS, dtype=jnp.float32)
    ylo = 1.0 - (2.0 * (tband * ROWS + ROWS - 1) + 1.0) / S - r_eps
    ys_seg = ys.reshape(b, nb, SEG)
    s_rel = (ys_seg[:, None, :, :] < ylo[None, :, None, None]).sum(
        axis=-1, dtype=jnp.int32)
    starts = ((s_rel // 8) * 8
              + (jnp.arange(nb, dtype=jnp.int32) * SW)[None, None, :])
    out = pl.pallas_call(
        _raster_body,
        grid=(b, NBANDS),
        in_specs=[
            pl.BlockSpec((1, 1, 1, nb), lambda bi, t: (bi, t, 0, 0),
                         memory_space=pltpu.SMEM),
            pl.BlockSpec((1, nt, 4), lambda bi, t: (bi, 0, 0)),
            pl.BlockSpec((1, 4, nt), lambda bi, t: (bi, 0, 0)),
            pl.BlockSpec((1, nt, c_dim), lambda bi, t: (bi, 0, 0)),
        ],
        out_specs=pl.BlockSpec((1, c_dim, PIX), lambda bi, t: (bi, 0, t)),
        out_shape=jax.ShapeDtypeStruct((b, c_dim, S * S), jnp.float32),
    )(starts.reshape(b, NBANDS, 1, nb), coords, coordst, featsp)
    return out.reshape(b, c_dim, S, S)
